# baseline (speedup 1.0000x reference)
import functools

import jax
import jax.numpy as jnp
from jax import lax
from jax.experimental import pallas as pl
from jax.experimental.pallas import tpu as pltpu
from jax.experimental.pallas import tpu_sc as plsc

_NUM_CORES = 2
_NUM_SUBCORES = 16
_NW = _NUM_CORES * _NUM_SUBCORES
_L = 16


@functools.partial(jax.jit, static_argnames=("n_tok", "seq_len", "d"))
def _emb_lookup(idx_flat, tok128, pos128, *, n_tok, seq_len, d):
    n_pairs_out = n_tok // 2
    pr_per_w = n_pairs_out // _NW
    pos_pairs = seq_len // 2
    chunk = seq_len
    n_segs = chunk // pos_pairs
    tok_per_chunk = 2 * chunk
    n_chunks = pr_per_w // chunk
    n_loop_pairs = n_chunks // 2
    d_vregs = d // _L
    full_vregs = tok_per_chunk // _L
    tail = tok_per_chunk - full_vregs * _L

    mesh = plsc.VectorSubcoreMesh(core_axis_name="c", subcore_axis_name="s")

    @functools.partial(
        pl.kernel,
        mesh=mesh,
        out_type=jax.ShapeDtypeStruct((n_pairs_out, 2 * d), jnp.float32),
        scratch_types=[
            pltpu.VMEM((tok_per_chunk + _L,), jnp.int32),
            pltpu.VMEM((tok_per_chunk + _L,), jnp.int32),
            pltpu.VMEM((tok_per_chunk,), jnp.int32),
            pltpu.VMEM((tok_per_chunk,), jnp.int32),
            pltpu.VMEM((tok_per_chunk, 2 * d), jnp.float32),
            pltpu.VMEM((tok_per_chunk, 2 * d), jnp.float32),
            pltpu.VMEM((pos_pairs, 2 * d), jnp.float32),
            pltpu.SemaphoreType.DMA,
            pltpu.SemaphoreType.DMA,
            pltpu.SemaphoreType.DMA,
            pltpu.SemaphoreType.DMA,
        ],
    )
    def body(idx_hbm, tok_hbm, pos_hbm, out_hbm,
             raw_a, raw_b, gl_a, gl_b, gb_a, gb_b, pos_v,
             gsem_a, gsem_b, ssem_a, ssem_b):
        cid = lax.axis_index("c")
        sid = lax.axis_index("s")
        wid = sid * _NUM_CORES + cid
        base_pr = wid * pr_per_w

        pltpu.sync_copy(pos_hbm, pos_v)

        lanes = lax.iota(jnp.int32, _L)
        tail_mask = lanes < tail

        def start_gather(g, raw_r, gl_r, gb_r, gsem):
            tok_start = (base_pr + g * chunk) * 2
            pltpu.sync_copy(idx_hbm.at[pl.ds(tok_start, tok_per_chunk)],
                            raw_r.at[pl.ds(0, tok_per_chunk)])
            for k in range(full_vregs):
                sl = pl.ds(k * _L, _L)
                gl_r[sl] = raw_r[sl] >> 1
            if tail:
                tidx = full_vregs * _L + lanes
                vals = plsc.load_gather(raw_r, [tidx], mask=tail_mask)
                plsc.store_scatter(gl_r, [tidx], vals >> 1, mask=tail_mask)
            pltpu.async_copy(tok_hbm.at[gl_r], gb_r, gsem)

        def wait_gather(gl_r, gb_r, gsem):
            pltpu.make_async_copy(tok_hbm.at[gl_r], gb_r, gsem).wait()

        def select_add(raw_r, gb_r):
            for seg in range(n_segs):
                def j_body(j, carry, seg=seg):
                    jj = seg * pos_pairs + j
                    pair_raw = raw_r[pl.ds(2 * jj, _L)]
                    for half in range(2):
                        r = 2 * jj + half
                        p = (pair_raw[half] & 1) * d
                        for c in range(d_vregs):
                            dst = pl.ds(half * d + c * _L, _L)
                            src = pl.ds(p + c * _L, _L)
                            gb_r[jj, dst] = gb_r[r, src] + pos_v[j, dst]
                    return carry
                lax.fori_loop(0, pos_pairs, j_body, 0, unroll=4)

        def start_store(g, gb_r, ssem):
            out_start = base_pr + g * chunk
            pltpu.async_copy(gb_r.at[pl.ds(0, chunk)],
                             out_hbm.at[pl.ds(out_start, chunk)], ssem)

        def wait_store(g, gb_r, ssem):
            out_start = base_pr + g * chunk
            pltpu.make_async_copy(gb_r.at[pl.ds(0, chunk)],
                                  out_hbm.at[pl.ds(out_start, chunk)],
                                  ssem).wait()

        start_gather(0, raw_a, gl_a, gb_a, gsem_a)

        def pair_body(h, carry):
            ga = 2 * h
            gb = 2 * h + 1

            @pl.when(h > 0)
            def _():
                wait_store(gb - 2, gb_b, ssem_b)
            start_gather(gb, raw_b, gl_b, gb_b, gsem_b)

            wait_gather(gl_a, gb_a, gsem_a)
            select_add(raw_a, gb_a)
            start_store(ga, gb_a, ssem_a)

            @pl.when(h + 1 < n_loop_pairs)
            def _():
                wait_store(ga, gb_a, ssem_a)
                start_gather(ga + 2, raw_a, gl_a, gb_a, gsem_a)

            wait_gather(gl_b, gb_b, gsem_b)
            select_add(raw_b, gb_b)
            start_store(gb, gb_b, ssem_b)
            return carry

        lax.fori_loop(0, n_loop_pairs, pair_body, 0)

        wait_store(n_chunks - 2, gb_a, ssem_a)
        wait_store(n_chunks - 1, gb_b, ssem_b)

    return body(idx_flat, tok128, pos128)


def kernel(inputs, token_table, pos_table):
    b, s = inputs.shape
    d = token_table.shape[1]
    idx_flat = inputs.reshape(-1).astype(jnp.int32)
    out = _emb_lookup(idx_flat, token_table.reshape(-1, 2 * d),
                      pos_table.reshape(-1, 2 * d),
                      n_tok=b * s, seq_len=s, d=d)
    return out.reshape(b, s, d)
